# baseline (speedup 1.0000x reference)
import functools

import jax
import jax.numpy as jnp
from jax import lax
from jax.experimental import pallas as pl
from jax.experimental.pallas import tpu as pltpu
from jax.experimental.pallas import tpu_sc as plsc

_E = 64
_D = 768
_H = 1024
_S = 2048

_ALIGN = 8
_BM = 64
_T = 96
_NW = 32
_P = 2816
_SLACK_ROW = 2688


def _router_body(x_ref, wg_ref, dest_ref, ex_ref, ts_ref):
    scores = lax.dot_general(x_ref[...], wg_ref[...],
                             (((1,), (1,)), ((), ())),
                             preferred_element_type=jnp.float32)
    idx = jnp.argmax(scores, axis=1).astype(jnp.int32)
    eids = lax.broadcasted_iota(jnp.int32, (1, _E), 1)
    onehot = (idx[:, None] == eids).astype(jnp.float32)
    r = lax.broadcasted_iota(jnp.int32, (_S, _S), 0)
    c = lax.broadcasted_iota(jnp.int32, (_S, _S), 1)
    tril = (c <= r).astype(jnp.float32)
    csum = lax.dot_general(tril, onehot, (((1,), (0,)), ((), ())),
                           preferred_element_type=jnp.float32)
    within = jnp.sum(csum * onehot, axis=1) - 1.0
    counts = csum[_S - 1:_S, :]
    padded = jnp.floor((counts + (_ALIGN - 1)) * (1.0 / _ALIGN)) * _ALIGN
    re = lax.broadcasted_iota(jnp.int32, (_E, _E), 0)
    ce = lax.broadcasted_iota(jnp.int32, (_E, _E), 1)
    sut = (re < ce).astype(jnp.float32)
    poff = lax.dot_general(padded, sut, (((1,), (0,)), ((), ())),
                           preferred_element_type=jnp.float32)
    off_tok = jnp.sum(onehot * poff, axis=1)
    dest_ref[...] = (off_tok + within).astype(jnp.int32)

    tiles = jnp.floor((counts + (_BM - 1)) * (1.0 / _BM))
    tbase = lax.dot_general(tiles, sut, (((1,), (0,)), ((), ())),
                            preferred_element_type=jnp.float32)
    tend = tbase + tiles
    tif = lax.broadcasted_iota(jnp.int32, (_T, _E), 0).astype(jnp.float32)
    exf = jnp.sum((tend <= tif).astype(jnp.float32), axis=1)
    eidsf = eids.astype(jnp.float32)
    emax = jnp.max(eidsf[0] * (counts[0] > 0.0).astype(jnp.float32))
    exc = jnp.minimum(exf, emax)
    ex_oh = (exc[:, None] == eidsf).astype(jnp.float32)
    tnum = lax.broadcasted_iota(jnp.int32, (_T,), 0).astype(jnp.float32)
    tstart = (jnp.sum(ex_oh * poff, axis=1)
              + (tnum - jnp.sum(ex_oh * tbase, axis=1)) * _BM)
    tstart = jnp.where(exf < float(_E), tstart, float(_SLACK_ROW))
    ex_ref[...] = exc.astype(jnp.int32)
    ts_ref[...] = tstart.astype(jnp.int32)


def _router(x_flat, wg):
    return pl.pallas_call(
        _router_body,
        out_shape=[
            jax.ShapeDtypeStruct((_S,), jnp.int32),
            jax.ShapeDtypeStruct((_T,), jnp.int32),
            jax.ShapeDtypeStruct((_T,), jnp.int32),
        ],
    )(x_flat, wg)


def _ffn_body(ex_ref, ts_ref, x_ref, w1_ref, w2_ref, w3_ref, out_ref):
    t = pl.program_id(0)
    row = pl.multiple_of(ts_ref[t], _ALIGN)
    xs = x_ref[pl.ds(row, _BM), :]
    w1 = w1_ref[0]
    w2 = w2_ref[0]
    w3 = w3_ref[0]
    h1 = lax.dot_general(xs, w1, (((1,), (1,)), ((), ())),
                         preferred_element_type=jnp.float32)
    h2 = lax.dot_general(xs, w2, (((1,), (1,)), ((), ())),
                         preferred_element_type=jnp.float32)
    h = h1 * jax.nn.sigmoid(h1) * h2
    o = lax.dot_general(h, w3, (((1,), (1,)), ((), ())),
                        preferred_element_type=jnp.float32)
    out_ref[pl.ds(row, _BM), :] = o


def _grouped_ffn(ex, tstart, x_sorted, w1, w2, w3):
    grid_spec = pltpu.PrefetchScalarGridSpec(
        num_scalar_prefetch=2,
        grid=(_T,),
        in_specs=[
            pl.BlockSpec((_P, _D), lambda t, ex, ts: (0, 0)),
            pl.BlockSpec((1, _H, _D), lambda t, ex, ts: (ex[t], 0, 0)),
            pl.BlockSpec((1, _H, _D), lambda t, ex, ts: (ex[t], 0, 0)),
            pl.BlockSpec((1, _D, _H), lambda t, ex, ts: (ex[t], 0, 0)),
        ],
        out_specs=pl.BlockSpec((_P, _D), lambda t, ex, ts: (0, 0)),
    )
    return pl.pallas_call(
        _ffn_body,
        grid_spec=grid_spec,
        out_shape=jax.ShapeDtypeStruct((_P, _D), jnp.float32),
    )(ex, tstart, x_sorted, w1, w2, w3)


def _sc_mesh():
    return plsc.VectorSubcoreMesh(core_axis_name="c", subcore_axis_name="s")


def _scatter_rows(x_flat, dest):
    n_per_w = _S // _NW

    @functools.partial(
        pl.kernel,
        out_type=jax.ShapeDtypeStruct((_P, _D), jnp.float32),
        mesh=_sc_mesh(),
        scratch_types=[
            pltpu.VMEM((n_per_w,), jnp.int32),
            pltpu.VMEM((n_per_w, _D), jnp.float32),
            pltpu.SemaphoreType.DMA,
        ],
    )
    def body(x_hbm, dest_hbm, out_hbm, idx_v, rows_v, sem):
        wid = lax.axis_index("s") * 2 + lax.axis_index("c")
        base = wid * n_per_w
        pltpu.sync_copy(dest_hbm.at[pl.ds(base, n_per_w)], idx_v)
        pltpu.sync_copy(x_hbm.at[pl.ds(base, n_per_w)], rows_v)
        pltpu.async_copy(rows_v, out_hbm.at[idx_v], sem).wait()

    return body(x_flat, dest)


def _gather_rows(table, dest):
    n_per_w = _S // _NW

    @functools.partial(
        pl.kernel,
        out_type=jax.ShapeDtypeStruct((_S, _D), jnp.float32),
        mesh=_sc_mesh(),
        scratch_types=[
            pltpu.VMEM((n_per_w,), jnp.int32),
            pltpu.VMEM((n_per_w, _D), jnp.float32),
            pltpu.SemaphoreType.DMA,
        ],
    )
    def body(table_hbm, dest_hbm, out_hbm, idx_v, rows_v, sem):
        wid = lax.axis_index("s") * 2 + lax.axis_index("c")
        base = wid * n_per_w
        pltpu.sync_copy(dest_hbm.at[pl.ds(base, n_per_w)], idx_v)
        pltpu.async_copy(table_hbm.at[idx_v], rows_v, sem).wait()
        pltpu.sync_copy(rows_v, out_hbm.at[pl.ds(base, n_per_w)])

    return body(table, dest)


def kernel(x, Wg, W1, W2, W3):
    b, s, d = x.shape
    x_flat = x.reshape(b * s, d)

    dest, ex, tstart = _router(x_flat, Wg)

    x_sorted = _scatter_rows(x_flat, dest)
    out_sorted = _grouped_ffn(ex, tstart, x_sorted, W1, W2, W3)
    out_flat = _gather_rows(out_sorted, dest)
    return out_flat.reshape(b, s, d)


# speedup vs baseline: 1.2281x; 1.2281x over previous
import functools

import jax
import jax.numpy as jnp
from jax import lax
from jax.experimental import pallas as pl
from jax.experimental.pallas import tpu as pltpu
from jax.experimental.pallas import tpu_sc as plsc

_E = 64
_D = 768
_H = 1024
_S = 2048

_ALIGN = 8
_BM = 64
_T = 96
_NW = 32
_P = 2816
_SLACK_ROW = 2688


def _router_body(x_ref, wg_ref, dest_ref, ex_ref, ts_ref):
    scores = lax.dot_general(x_ref[...], wg_ref[...],
                             (((1,), (1,)), ((), ())),
                             preferred_element_type=jnp.float32)
    idx = jnp.argmax(scores, axis=1).astype(jnp.int32)
    eids = lax.broadcasted_iota(jnp.int32, (1, _E), 1)
    onehot = (idx[:, None] == eids).astype(jnp.float32)
    r = lax.broadcasted_iota(jnp.int32, (_S, _S), 0)
    c = lax.broadcasted_iota(jnp.int32, (_S, _S), 1)
    tril = (c <= r).astype(jnp.float32)
    csum = lax.dot_general(tril, onehot, (((1,), (0,)), ((), ())),
                           preferred_element_type=jnp.float32)
    within = jnp.sum(csum * onehot, axis=1) - 1.0
    counts = csum[_S - 1:_S, :]
    padded = jnp.floor((counts + (_ALIGN - 1)) * (1.0 / _ALIGN)) * _ALIGN
    re = lax.broadcasted_iota(jnp.int32, (_E, _E), 0)
    ce = lax.broadcasted_iota(jnp.int32, (_E, _E), 1)
    sut = (re < ce).astype(jnp.float32)
    poff = lax.dot_general(padded, sut, (((1,), (0,)), ((), ())),
                           preferred_element_type=jnp.float32)
    off_tok = jnp.sum(onehot * poff, axis=1)
    dest_ref[...] = (off_tok + within).astype(jnp.int32)

    tiles = jnp.floor((counts + (_BM - 1)) * (1.0 / _BM))
    tbase = lax.dot_general(tiles, sut, (((1,), (0,)), ((), ())),
                            preferred_element_type=jnp.float32)
    tend = tbase + tiles
    tif = lax.broadcasted_iota(jnp.int32, (_T, _E), 0).astype(jnp.float32)
    exf = jnp.sum((tend <= tif).astype(jnp.float32), axis=1)
    eidsf = eids.astype(jnp.float32)
    emax = jnp.max(eidsf[0] * (counts[0] > 0.0).astype(jnp.float32))
    exc = jnp.minimum(exf, emax)
    ex_oh = (exc[:, None] == eidsf).astype(jnp.float32)
    tnum = lax.broadcasted_iota(jnp.int32, (_T,), 0).astype(jnp.float32)
    tstart = (jnp.sum(ex_oh * poff, axis=1)
              + (tnum - jnp.sum(ex_oh * tbase, axis=1)) * _BM)
    tstart = jnp.where(exf < float(_E), tstart, float(_SLACK_ROW))
    ex_ref[...] = exc.astype(jnp.int32)
    ts_ref[...] = tstart.astype(jnp.int32)


def _router(x_flat, wg):
    return pl.pallas_call(
        _router_body,
        out_shape=[
            jax.ShapeDtypeStruct((_S,), jnp.int32),
            jax.ShapeDtypeStruct((_T,), jnp.int32),
            jax.ShapeDtypeStruct((_T,), jnp.int32),
        ],
    )(x_flat, wg)


def _ffn_body(ex_ref, ts_ref, x_ref, w1_ref, w2_ref, w3_ref, out_ref):
    t = pl.program_id(0)
    ts = ts_ref[t]

    @pl.when(ts < _SLACK_ROW)
    def _():
        row = pl.multiple_of(ts, _ALIGN)
        xs = x_ref[pl.ds(row, _BM), :]
        w1 = w1_ref[0]
        w2 = w2_ref[0]
        w3 = w3_ref[0]
        h1 = lax.dot_general(xs, w1, (((1,), (1,)), ((), ())),
                             preferred_element_type=jnp.float32)
        h2 = lax.dot_general(xs, w2, (((1,), (1,)), ((), ())),
                             preferred_element_type=jnp.float32)
        h = h1 * jax.nn.sigmoid(h1) * h2
        o = lax.dot_general(h, w3, (((1,), (1,)), ((), ())),
                            preferred_element_type=jnp.float32)
        out_ref[pl.ds(row, _BM), :] = o


def _grouped_ffn(ex, tstart, x_sorted, w1, w2, w3):
    grid_spec = pltpu.PrefetchScalarGridSpec(
        num_scalar_prefetch=2,
        grid=(_T,),
        in_specs=[
            pl.BlockSpec((_P, _D), lambda t, ex, ts: (0, 0)),
            pl.BlockSpec((1, _H, _D), lambda t, ex, ts: (ex[t], 0, 0)),
            pl.BlockSpec((1, _H, _D), lambda t, ex, ts: (ex[t], 0, 0)),
            pl.BlockSpec((1, _D, _H), lambda t, ex, ts: (ex[t], 0, 0)),
        ],
        out_specs=pl.BlockSpec((_P, _D), lambda t, ex, ts: (0, 0)),
    )
    return pl.pallas_call(
        _ffn_body,
        grid_spec=grid_spec,
        out_shape=jax.ShapeDtypeStruct((_P, _D), jnp.float32),
    )(ex, tstart, x_sorted, w1, w2, w3)


def _sc_mesh():
    return plsc.VectorSubcoreMesh(core_axis_name="c", subcore_axis_name="s")


def _scatter_rows(x_flat, dest):
    n_per_w = _S // _NW

    @functools.partial(
        pl.kernel,
        out_type=jax.ShapeDtypeStruct((_P, _D), jnp.float32),
        mesh=_sc_mesh(),
        scratch_types=[
            pltpu.VMEM((n_per_w,), jnp.int32),
            pltpu.VMEM((n_per_w, _D), jnp.float32),
            pltpu.SemaphoreType.DMA,
        ],
    )
    def body(x_hbm, dest_hbm, out_hbm, idx_v, rows_v, sem):
        wid = lax.axis_index("s") * 2 + lax.axis_index("c")
        base = wid * n_per_w
        pltpu.sync_copy(dest_hbm.at[pl.ds(base, n_per_w)], idx_v)
        pltpu.sync_copy(x_hbm.at[pl.ds(base, n_per_w)], rows_v)
        pltpu.async_copy(rows_v, out_hbm.at[idx_v], sem).wait()

    return body(x_flat, dest)


def _gather_rows(table, dest):
    n_per_w = _S // _NW

    @functools.partial(
        pl.kernel,
        out_type=jax.ShapeDtypeStruct((_S, _D), jnp.float32),
        mesh=_sc_mesh(),
        scratch_types=[
            pltpu.VMEM((n_per_w,), jnp.int32),
            pltpu.VMEM((n_per_w, _D), jnp.float32),
            pltpu.SemaphoreType.DMA,
        ],
    )
    def body(table_hbm, dest_hbm, out_hbm, idx_v, rows_v, sem):
        wid = lax.axis_index("s") * 2 + lax.axis_index("c")
        base = wid * n_per_w
        pltpu.sync_copy(dest_hbm.at[pl.ds(base, n_per_w)], idx_v)
        pltpu.async_copy(table_hbm.at[idx_v], rows_v, sem).wait()
        pltpu.sync_copy(rows_v, out_hbm.at[pl.ds(base, n_per_w)])

    return body(table, dest)


def kernel(x, Wg, W1, W2, W3):
    b, s, d = x.shape
    x_flat = x.reshape(b * s, d)

    dest, ex, tstart = _router(x_flat, Wg)

    x_sorted = _scatter_rows(x_flat, dest)
    out_sorted = _grouped_ffn(ex, tstart, x_sorted, W1, W2, W3)
    out_flat = _gather_rows(out_sorted, dest)
    return out_flat.reshape(b, s, d)
